# baseline (device time: 202293 ns/iter reference)
import jax
import jax.numpy as jnp
from jax import lax
from jax.experimental import pallas as pl
from jax.experimental.pallas import tpu as pltpu

BK = 512


def _flash_body(y_ref, q_ref, k_ref, v_ref, o_ref, m_ref, l_ref, acc, m_s, l_s):
    del y_ref
    kc = pl.program_id(1)
    n_kc = pl.num_programs(1)
    n_heads = q_ref.shape[2] // 128

    @pl.when(kc == 0)
    def _():
        acc[...] = jnp.zeros_like(acc)
        m_s[...] = jnp.full_like(m_s, -jnp.inf)
        l_s[...] = jnp.zeros_like(l_s)

    for hi in range(n_heads):
        sl = pl.ds(hi * 128, 128)
        q = q_ref[0, :, sl].astype(jnp.bfloat16)
        k = k_ref[0, :, sl].astype(jnp.bfloat16)
        v = v_ref[0, :, sl].astype(jnp.bfloat16)

        s = lax.dot_general(
            q, k, (((1,), (1,)), ((), ())), preferred_element_type=jnp.float32
        ) * (128.0 ** -0.5)

        m_prev = m_s[hi]
        m_cur = jnp.max(s, axis=1, keepdims=True)
        m_new = jnp.maximum(m_prev, m_cur)
        alpha = jnp.exp(m_prev - m_new)
        p = jnp.exp(s - m_new[:, 0:1])
        l_s[hi] = alpha * l_s[hi] + jnp.sum(p, axis=1, keepdims=True)
        m_s[hi] = m_new
        acc[hi] = acc[hi] * alpha[:, 0:1] + lax.dot_general(
            p.astype(jnp.bfloat16), v, (((1,), (0,)), ((), ())),
            preferred_element_type=jnp.float32,
        )

    @pl.when(kc == n_kc - 1)
    def _():
        for hi in range(n_heads):
            o_ref[0, :, pl.ds(hi * 128, 128)] = acc[hi].astype(o_ref.dtype)
            m_ref[0, :, hi] = m_s[hi, :, 0]
            l_ref[0, :, hi] = l_s[hi, :, 0]


def _merge(m_a, l_a, o_a, m_b, l_b, o_b):
    m_new = jnp.maximum(m_a, m_b)
    a = jnp.exp(m_a - m_new)
    b = jnp.exp(m_b - m_new)
    l_new = a * l_a + b * l_b
    o_new = o_a * a[:, :, :, None] + o_b * b[:, :, :, None]
    return m_new, l_new, o_new


def _combine_body(o_ref, m_ref, l_ref, out_ref,
                  comm1_o, comm1_m, comm1_l,
                  send2_o, send2_m, send2_l,
                  comm2_o, comm2_m, comm2_l,
                  send_sems, recv_sems):
    my_x = lax.axis_index("x")
    my_y = lax.axis_index("y")
    y_partner = (my_x, 1 - my_y)
    x_partner = (1 - my_x, my_y)

    barrier = pltpu.get_barrier_semaphore()
    for nbr in (y_partner, x_partner):
        pl.semaphore_signal(barrier, inc=1, device_id=nbr,
                            device_id_type=pl.DeviceIdType.MESH)
    pl.semaphore_wait(barrier, 2)

    stage1 = []
    for i, (src, dst) in enumerate(
        ((o_ref, comm1_o), (m_ref, comm1_m), (l_ref, comm1_l))
    ):
        c = pltpu.make_async_remote_copy(
            src_ref=src, dst_ref=dst,
            send_sem=send_sems.at[i], recv_sem=recv_sems.at[i],
            device_id=y_partner, device_id_type=pl.DeviceIdType.MESH,
        )
        c.start()
        stage1.append(c)
    for c in stage1:
        c.wait()

    m1, l1, o1 = _merge(
        m_ref[...], l_ref[...], o_ref[...].astype(jnp.float32),
        comm1_m[...], comm1_l[...], comm1_o[...].astype(jnp.float32),
    )
    send2_o[...] = o1.astype(jnp.bfloat16)
    send2_m[...] = m1
    send2_l[...] = l1

    stage2 = []
    for i, (src, dst) in enumerate(
        ((send2_o, comm2_o), (send2_m, comm2_m), (send2_l, comm2_l))
    ):
        c = pltpu.make_async_remote_copy(
            src_ref=src, dst_ref=dst,
            send_sem=send_sems.at[3 + i], recv_sem=recv_sems.at[3 + i],
            device_id=x_partner, device_id_type=pl.DeviceIdType.MESH,
        )
        c.start()
        stage2.append(c)
    for c in stage2:
        c.wait()

    _, l_f, o_f = _merge(
        m1, l1, send2_o[...].astype(jnp.float32),
        comm2_m[...], comm2_l[...], comm2_o[...].astype(jnp.float32),
    )
    out_ref[...] = o_f / l_f[:, :, :, None]


def kernel(Q, K, V):
    b, q_len, h, d = Q.shape
    kv_len = K.shape[1]
    hd = h * d
    half = kv_len // 2
    n_kc = half // BK

    Qc = Q.reshape(b, q_len, hd)
    Kc = K.reshape(b, kv_len, hd)
    Vc = V.reshape(b, kv_len, hd)

    y_arr = jnp.reshape(lax.axis_index("y"), (1,)).astype(jnp.int32)

    def kv_map(bi, kc, y_sp):
        return (bi, y_sp[0] * n_kc + kc, 0)

    grid_spec = pltpu.PrefetchScalarGridSpec(
        num_scalar_prefetch=1,
        grid=(b, n_kc),
        in_specs=[
            pl.BlockSpec((1, q_len, hd), lambda bi, kc, y_sp: (bi, 0, 0)),
            pl.BlockSpec((1, BK, hd), kv_map),
            pl.BlockSpec((1, BK, hd), kv_map),
        ],
        out_specs=[
            pl.BlockSpec((1, q_len, hd), lambda bi, kc, y_sp: (bi, 0, 0)),
            pl.BlockSpec((1, q_len, h), lambda bi, kc, y_sp: (bi, 0, 0)),
            pl.BlockSpec((1, q_len, h), lambda bi, kc, y_sp: (bi, 0, 0)),
        ],
        scratch_shapes=[
            pltpu.VMEM((h, q_len, d), jnp.float32),
            pltpu.VMEM((h, q_len, d), jnp.float32),
            pltpu.VMEM((h, q_len, d), jnp.float32),
        ],
    )
    o_part, m_part, l_part = pl.pallas_call(
        _flash_body,
        grid_spec=grid_spec,
        out_shape=[
            jax.ShapeDtypeStruct((b, q_len, hd), jnp.bfloat16),
            jax.ShapeDtypeStruct((b, q_len, h), jnp.float32),
            jax.ShapeDtypeStruct((b, q_len, h), jnp.float32),
        ],
    )(y_arr, Qc, Kc, Vc)
    o_part = o_part.reshape(b, q_len, h, d)

    return pl.pallas_call(
        _combine_body,
        in_specs=[pl.BlockSpec(memory_space=pltpu.VMEM)] * 3,
        out_specs=pl.BlockSpec(memory_space=pltpu.VMEM),
        out_shape=jax.ShapeDtypeStruct((b, q_len, h, d), jnp.float32),
        scratch_shapes=[
            pltpu.VMEM((b, q_len, h, d), jnp.bfloat16),
            pltpu.VMEM((b, q_len, h), jnp.float32),
            pltpu.VMEM((b, q_len, h), jnp.float32),
            pltpu.VMEM((b, q_len, h, d), jnp.bfloat16),
            pltpu.VMEM((b, q_len, h), jnp.float32),
            pltpu.VMEM((b, q_len, h), jnp.float32),
            pltpu.VMEM((b, q_len, h, d), jnp.bfloat16),
            pltpu.VMEM((b, q_len, h), jnp.float32),
            pltpu.VMEM((b, q_len, h), jnp.float32),
            pltpu.SemaphoreType.DMA((6,)),
            pltpu.SemaphoreType.DMA((6,)),
        ],
        compiler_params=pltpu.CompilerParams(collective_id=0),
    )(o_part, m_part, l_part)


# device time: 47558 ns/iter; 4.2536x vs baseline; 4.2536x over previous
import jax
import jax.numpy as jnp
from jax import lax
from jax.experimental import pallas as pl
from jax.experimental.pallas import tpu as pltpu

BK = 1024
NSLOT = 4


def _make_flash_body(b, q_len, h, d, half, n_kc):
    steps = [(bi, hi, kc) for bi in range(b) for hi in range(h)
             for kc in range(n_kc)]
    T = len(steps)

    def body(q_ref, k_hbm, v_hbm, o_ref, m_ref, l_ref,
             kbuf, vbuf, sem_k, sem_v):
        my_y = lax.axis_index("y")
        row_base = my_y * half

        dmas = {}

        def issue(t):
            bi, hi, kc = steps[t]
            slot = t % NSLOT
            rows = pl.ds(row_base + kc * BK, BK)
            ck = pltpu.make_async_copy(
                k_hbm.at[bi, rows, hi, :], kbuf.at[slot], sem_k.at[slot])
            cv = pltpu.make_async_copy(
                v_hbm.at[bi, rows, hi, :], vbuf.at[slot], sem_v.at[slot])
            ck.start()
            cv.start()
            dmas[t] = (ck, cv)

        for t in range(min(NSLOT - 1, T)):
            issue(t)

        state = None
        for t in range(T):
            bi, hi, kc = steps[t]
            if t + NSLOT - 1 < T:
                issue(t + NSLOT - 1)
            ck, cv = dmas.pop(t)
            ck.wait()
            cv.wait()
            slot = t % NSLOT

            q = q_ref[bi, :, pl.ds(hi * d, d)].astype(jnp.bfloat16)
            k = kbuf[slot].astype(jnp.bfloat16)
            v = vbuf[slot].astype(jnp.bfloat16)

            s = lax.dot_general(
                q, k, (((1,), (1,)), ((), ())),
                preferred_element_type=jnp.float32,
            ) * (128.0 ** -0.5)
            m_cur = jnp.max(s, axis=1, keepdims=True)

            if kc == 0:
                m_new = m_cur
                p = jnp.exp(s - m_new)
                l_new = jnp.sum(p, axis=1, keepdims=True)
                acc = lax.dot_general(
                    p.astype(jnp.bfloat16), v, (((1,), (0,)), ((), ())),
                    preferred_element_type=jnp.float32,
                )
            else:
                m_prev, l_prev, acc_prev = state
                m_new = jnp.maximum(m_prev, m_cur)
                alpha = jnp.exp(m_prev - m_new)
                p = jnp.exp(s - m_new)
                l_new = alpha * l_prev + jnp.sum(p, axis=1, keepdims=True)
                acc = acc_prev * alpha + lax.dot_general(
                    p.astype(jnp.bfloat16), v, (((1,), (0,)), ((), ())),
                    preferred_element_type=jnp.float32,
                )
            state = (m_new, l_new, acc)

            if kc == n_kc - 1:
                o_ref[bi, :, pl.ds(hi * d, d)] = acc.astype(o_ref.dtype)
                m_ref[bi, :, hi] = m_new[:, 0]
                l_ref[bi, :, hi] = l_new[:, 0]

    return body


def _merge(m_a, l_a, o_a, m_b, l_b, o_b):
    m_new = jnp.maximum(m_a, m_b)
    a = jnp.exp(m_a - m_new)
    b = jnp.exp(m_b - m_new)
    l_new = a * l_a + b * l_b
    o_new = o_a * a[:, :, :, None] + o_b * b[:, :, :, None]
    return m_new, l_new, o_new


def _combine_body(o_ref, m_ref, l_ref, out_ref,
                  comm1_o, comm1_m, comm1_l,
                  send2_o, send2_m, send2_l,
                  comm2_o, comm2_m, comm2_l,
                  send_sems, recv_sems):
    my_x = lax.axis_index("x")
    my_y = lax.axis_index("y")
    y_partner = (my_x, 1 - my_y)
    x_partner = (1 - my_x, my_y)

    barrier = pltpu.get_barrier_semaphore()
    for nbr in (y_partner, x_partner):
        pl.semaphore_signal(barrier, inc=1, device_id=nbr,
                            device_id_type=pl.DeviceIdType.MESH)
    pl.semaphore_wait(barrier, 2)

    stage1 = []
    for i, (src, dst) in enumerate(
        ((o_ref, comm1_o), (m_ref, comm1_m), (l_ref, comm1_l))
    ):
        c = pltpu.make_async_remote_copy(
            src_ref=src, dst_ref=dst,
            send_sem=send_sems.at[i], recv_sem=recv_sems.at[i],
            device_id=y_partner, device_id_type=pl.DeviceIdType.MESH,
        )
        c.start()
        stage1.append(c)
    for c in stage1:
        c.wait()

    m1, l1, o1 = _merge(
        m_ref[...], l_ref[...], o_ref[...].astype(jnp.float32),
        comm1_m[...], comm1_l[...], comm1_o[...].astype(jnp.float32),
    )
    send2_o[...] = o1.astype(jnp.bfloat16)
    send2_m[...] = m1
    send2_l[...] = l1

    stage2 = []
    for i, (src, dst) in enumerate(
        ((send2_o, comm2_o), (send2_m, comm2_m), (send2_l, comm2_l))
    ):
        c = pltpu.make_async_remote_copy(
            src_ref=src, dst_ref=dst,
            send_sem=send_sems.at[3 + i], recv_sem=recv_sems.at[3 + i],
            device_id=x_partner, device_id_type=pl.DeviceIdType.MESH,
        )
        c.start()
        stage2.append(c)
    for c in stage2:
        c.wait()

    _, l_f, o_f = _merge(
        m1, l1, send2_o[...].astype(jnp.float32),
        comm2_m[...], comm2_l[...], comm2_o[...].astype(jnp.float32),
    )
    out_ref[...] = o_f / l_f[:, :, :, None]


def kernel(Q, K, V):
    b, q_len, h, d = Q.shape
    kv_len = K.shape[1]
    hd = h * d
    half = kv_len // 2
    n_kc = half // BK

    Qc = Q.reshape(b, q_len, hd)

    o_part, m_part, l_part = pl.pallas_call(
        _make_flash_body(b, q_len, h, d, half, n_kc),
        in_specs=[
            pl.BlockSpec(memory_space=pltpu.VMEM),
            pl.BlockSpec(memory_space=pl.ANY),
            pl.BlockSpec(memory_space=pl.ANY),
        ],
        out_specs=[pl.BlockSpec(memory_space=pltpu.VMEM)] * 3,
        out_shape=[
            jax.ShapeDtypeStruct((b, q_len, hd), jnp.bfloat16),
            jax.ShapeDtypeStruct((b, q_len, h), jnp.float32),
            jax.ShapeDtypeStruct((b, q_len, h), jnp.float32),
        ],
        scratch_shapes=[
            pltpu.VMEM((NSLOT, BK, d), jnp.float32),
            pltpu.VMEM((NSLOT, BK, d), jnp.float32),
            pltpu.SemaphoreType.DMA((NSLOT,)),
            pltpu.SemaphoreType.DMA((NSLOT,)),
        ],
    )(Qc, K, V)
    o_part = o_part.reshape(b, q_len, h, d)

    return pl.pallas_call(
        _combine_body,
        in_specs=[pl.BlockSpec(memory_space=pltpu.VMEM)] * 3,
        out_specs=pl.BlockSpec(memory_space=pltpu.VMEM),
        out_shape=jax.ShapeDtypeStruct((b, q_len, h, d), jnp.float32),
        scratch_shapes=[
            pltpu.VMEM((b, q_len, h, d), jnp.bfloat16),
            pltpu.VMEM((b, q_len, h), jnp.float32),
            pltpu.VMEM((b, q_len, h), jnp.float32),
            pltpu.VMEM((b, q_len, h, d), jnp.bfloat16),
            pltpu.VMEM((b, q_len, h), jnp.float32),
            pltpu.VMEM((b, q_len, h), jnp.float32),
            pltpu.VMEM((b, q_len, h, d), jnp.bfloat16),
            pltpu.VMEM((b, q_len, h), jnp.float32),
            pltpu.VMEM((b, q_len, h), jnp.float32),
            pltpu.SemaphoreType.DMA((6,)),
            pltpu.SemaphoreType.DMA((6,)),
        ],
        compiler_params=pltpu.CompilerParams(collective_id=0),
    )(o_part, m_part, l_part)


# device time: 40791 ns/iter; 4.9593x vs baseline; 1.1659x over previous
import jax
import jax.numpy as jnp
from jax import lax
from jax.experimental import pallas as pl
from jax.experimental.pallas import tpu as pltpu

BK = 1024
NSLOT = 4


def _merge_h(m_a, l_a, o_a, m_b, l_b, o_b):
    m_new = jnp.maximum(m_a, m_b)
    a = jnp.exp(m_a - m_new)
    c = jnp.exp(m_b - m_new)
    return m_new, a * l_a + c * l_b, o_a * a + o_b * c


def _make_body(b, q_len, h, d, half, n_kc):
    steps = [(bi, hi, kc) for bi in range(b) for hi in range(h)
             for kc in range(n_kc)]
    T = len(steps)

    def body(q_ref, k_hbm, v_hbm, out_ref,
             kbuf, vbuf, sem_k, sem_v,
             part_o, part_m, part_l,
             comm1_o, comm1_m, comm1_l,
             send2_o, send2_m, send2_l,
             comm2_o, comm2_m, comm2_l,
             s1_send, s1_recv, s2_send, s2_recv):
        my_x = lax.axis_index("x")
        my_y = lax.axis_index("y")
        y_partner = (my_x, 1 - my_y)
        x_partner = (1 - my_x, my_y)
        row_base = my_y * half

        barrier = pltpu.get_barrier_semaphore()
        for nbr in (y_partner, x_partner):
            pl.semaphore_signal(barrier, inc=1, device_id=nbr,
                                device_id_type=pl.DeviceIdType.MESH)

        send_descs = []
        recv1 = {bi: [] for bi in range(b)}
        recv2 = {bi: [] for bi in range(b)}

        def exchange(bi, triples, sems, partner, recvs):
            send_sems, recv_sems = sems
            for j, (src, dst) in enumerate(triples):
                c = pltpu.make_async_remote_copy(
                    src_ref=src.at[bi], dst_ref=dst.at[bi],
                    send_sem=send_sems.at[bi, j], recv_sem=recv_sems.at[bi, j],
                    device_id=partner, device_id_type=pl.DeviceIdType.MESH,
                )
                c.start()
                send_descs.append(c)
                recvs[bi].append(c)

        dmas = {}

        def issue(t):
            bi, hi, kc = steps[t]
            slot = t % NSLOT
            rows = pl.ds(row_base + kc * BK, BK)
            ck = pltpu.make_async_copy(
                k_hbm.at[bi, rows, hi, :], kbuf.at[slot], sem_k.at[slot])
            cv = pltpu.make_async_copy(
                v_hbm.at[bi, rows, hi, :], vbuf.at[slot], sem_v.at[slot])
            ck.start()
            cv.start()
            dmas[t] = (ck, cv)

        for t in range(min(NSLOT - 1, T)):
            issue(t)

        state = None
        for t in range(T):
            bi, hi, kc = steps[t]
            if t + NSLOT - 1 < T:
                issue(t + NSLOT - 1)
            ck, cv = dmas.pop(t)
            ck.wait()
            cv.wait()
            slot = t % NSLOT

            q = q_ref[bi, :, pl.ds(hi * d, d)].astype(jnp.bfloat16)
            k = kbuf[slot].astype(jnp.bfloat16)
            v = vbuf[slot].astype(jnp.bfloat16)

            s = lax.dot_general(
                q, k, (((1,), (1,)), ((), ())),
                preferred_element_type=jnp.float32,
            ) * (128.0 ** -0.5)
            m_cur = jnp.max(s, axis=1, keepdims=True)

            if kc == 0:
                m_new = m_cur
                p = jnp.exp(s - m_new)
                l_new = jnp.sum(p, axis=1, keepdims=True)
                acc = lax.dot_general(
                    p.astype(jnp.bfloat16), v, (((1,), (0,)), ((), ())),
                    preferred_element_type=jnp.float32,
                )
            else:
                m_prev, l_prev, acc_prev = state
                m_new = jnp.maximum(m_prev, m_cur)
                alpha = jnp.exp(m_prev - m_new)
                p = jnp.exp(s - m_new)
                l_new = alpha * l_prev + jnp.sum(p, axis=1, keepdims=True)
                acc = acc_prev * alpha + lax.dot_general(
                    p.astype(jnp.bfloat16), v, (((1,), (0,)), ((), ())),
                    preferred_element_type=jnp.float32,
                )
            state = (m_new, l_new, acc)

            if kc == n_kc - 1:
                part_o[bi, :, pl.ds(hi * d, d)] = acc.astype(part_o.dtype)
                part_m[bi, :, hi] = m_new[:, 0]
                part_l[bi, :, hi] = l_new[:, 0]
                if hi == h - 1:
                    if bi == 0:
                        pl.semaphore_wait(barrier, 2)
                    exchange(bi,
                             ((part_o, comm1_o), (part_m, comm1_m),
                              (part_l, comm1_l)),
                             (s1_send, s1_recv), y_partner, recv1)

        for bi in range(b):
            for c in recv1[bi]:
                c.wait_recv()
            for hi in range(h):
                sl = pl.ds(hi * d, d)
                m1, l1, o1 = _merge_h(
                    part_m[bi][:, hi:hi + 1], part_l[bi][:, hi:hi + 1],
                    part_o[bi, :, sl].astype(jnp.float32),
                    comm1_m[bi][:, hi:hi + 1], comm1_l[bi][:, hi:hi + 1],
                    comm1_o[bi, :, sl].astype(jnp.float32),
                )
                send2_o[bi, :, sl] = o1.astype(send2_o.dtype)
                send2_m[bi, :, hi] = m1[:, 0]
                send2_l[bi, :, hi] = l1[:, 0]
            exchange(bi,
                     ((send2_o, comm2_o), (send2_m, comm2_m),
                      (send2_l, comm2_l)),
                     (s2_send, s2_recv), x_partner, recv2)

        for bi in range(b):
            for c in recv2[bi]:
                c.wait_recv()
            for hi in range(h):
                sl = pl.ds(hi * d, d)
                _, l_f, o_f = _merge_h(
                    send2_m[bi][:, hi:hi + 1], send2_l[bi][:, hi:hi + 1],
                    send2_o[bi, :, sl].astype(jnp.float32),
                    comm2_m[bi][:, hi:hi + 1], comm2_l[bi][:, hi:hi + 1],
                    comm2_o[bi, :, sl].astype(jnp.float32),
                )
                out_ref[bi, :, hi, :] = o_f / l_f

        for c in send_descs:
            c.wait_send()

    return body


def kernel(Q, K, V):
    b, q_len, h, d = Q.shape
    kv_len = K.shape[1]
    hd = h * d
    half = kv_len // 2
    n_kc = half // BK

    Qc = Q.reshape(b, q_len, hd)

    return pl.pallas_call(
        _make_body(b, q_len, h, d, half, n_kc),
        in_specs=[
            pl.BlockSpec(memory_space=pltpu.VMEM),
            pl.BlockSpec(memory_space=pl.ANY),
            pl.BlockSpec(memory_space=pl.ANY),
        ],
        out_specs=pl.BlockSpec(memory_space=pltpu.VMEM),
        out_shape=jax.ShapeDtypeStruct((b, q_len, h, d), jnp.float32),
        scratch_shapes=[
            pltpu.VMEM((NSLOT, BK, d), jnp.float32),
            pltpu.VMEM((NSLOT, BK, d), jnp.float32),
            pltpu.SemaphoreType.DMA((NSLOT,)),
            pltpu.SemaphoreType.DMA((NSLOT,)),
            pltpu.VMEM((b, q_len, hd), jnp.bfloat16),
            pltpu.VMEM((b, q_len, h), jnp.float32),
            pltpu.VMEM((b, q_len, h), jnp.float32),
            pltpu.VMEM((b, q_len, hd), jnp.bfloat16),
            pltpu.VMEM((b, q_len, h), jnp.float32),
            pltpu.VMEM((b, q_len, h), jnp.float32),
            pltpu.VMEM((b, q_len, hd), jnp.bfloat16),
            pltpu.VMEM((b, q_len, h), jnp.float32),
            pltpu.VMEM((b, q_len, h), jnp.float32),
            pltpu.VMEM((b, q_len, hd), jnp.bfloat16),
            pltpu.VMEM((b, q_len, h), jnp.float32),
            pltpu.VMEM((b, q_len, h), jnp.float32),
            pltpu.SemaphoreType.DMA((b, 3)),
            pltpu.SemaphoreType.DMA((b, 3)),
            pltpu.SemaphoreType.DMA((b, 3)),
            pltpu.SemaphoreType.DMA((b, 3)),
        ],
        compiler_params=pltpu.CompilerParams(collective_id=0),
    )(Qc, K, V)
